# Initial kernel scaffold; baseline (speedup 1.0000x reference)
import functools

import jax
import jax.numpy as jnp
from jax import lax
from jax.experimental import pallas as pl
from jax.experimental.pallas import tpu as pltpu
from jax.experimental.pallas import tpu_sc as plsc

N = 10000
E = 320000
E2 = 640000
D_IN = 128
HEADS = 8
C_HID = 32
D_MID = 256
D_OUT = 64
BN = 1000
F32 = jnp.float32


def _w_kernel(a_ref, w_ref):
    a = a_ref[...]
    m = jnp.max(a)
    e = jnp.exp(a - m)
    w_ref[...] = e / jnp.sum(e)


def _hop_weights(hop_attention):
    w2d = pl.pallas_call(
        _w_kernel,
        out_shape=jax.ShapeDtypeStruct((1, 2), F32),
    )(hop_attention.reshape(1, 2))
    return w2d


def _tc_pre_kernel(x_ref, w0_ref, w1_ref, wsp_ref, a0s_ref, a0d_ref, a1s_ref,
                   a1d_ref, h0c_ref, h1c_ref, xsp_ref, as0_ref, ad0_ref,
                   as1_ref, ad1_ref):
    xb = x_ref[...]
    z = jnp.zeros((BN, 8), F32)

    def hop(w_ref, asr, adr, hc_ref, aso_ref, ado_ref):
        h = jnp.dot(xb, w_ref[...], preferred_element_type=F32)
        hc_ref[0] = h[:, :128]
        hc_ref[1] = h[:, 128:]
        asv = (h * asr[...]).reshape(BN, HEADS, C_HID).sum(-1)
        adv = (h * adr[...]).reshape(BN, HEADS, C_HID).sum(-1)
        aso_ref[...] = jnp.concatenate([asv, z], axis=1)
        ado_ref[...] = jnp.concatenate([adv, z], axis=1)

    hop(w0_ref, a0s_ref, a0d_ref, h0c_ref, as0_ref, ad0_ref)
    hop(w1_ref, a1s_ref, a1d_ref, h1c_ref, as1_ref, ad1_ref)
    xsp_ref[...] = jnp.dot(xb, wsp_ref[...], preferred_element_type=F32)


def _tc_pre(x, W0, W1, Wsp, a_src0, a_dst0, a_src1, a_dst1):
    full_w = pl.BlockSpec((D_IN, D_MID), lambda i: (0, 0))
    full_a = pl.BlockSpec((1, D_MID), lambda i: (0, 0))
    row = pl.BlockSpec((BN, D_IN), lambda i: (i, 0))
    chunk_o = pl.BlockSpec((2, BN, 128), lambda i: (0, i, 0))
    tbl_o = pl.BlockSpec((BN, 16), lambda i: (i, 0))
    return pl.pallas_call(
        _tc_pre_kernel,
        grid=(N // BN,),
        in_specs=[row, full_w, full_w, full_w, full_a, full_a, full_a, full_a],
        out_specs=[chunk_o, chunk_o, pl.BlockSpec((BN, D_MID), lambda i: (i, 0)),
                   tbl_o, tbl_o, tbl_o, tbl_o],
        out_shape=[
            jax.ShapeDtypeStruct((2, N, 128), F32),
            jax.ShapeDtypeStruct((2, N, 128), F32),
            jax.ShapeDtypeStruct((N, D_MID), F32),
            jax.ShapeDtypeStruct((N, 16), F32),
            jax.ShapeDtypeStruct((N, 16), F32),
            jax.ShapeDtypeStruct((N, 16), F32),
            jax.ShapeDtypeStruct((N, 16), F32),
        ],
    )(x, W0, W1, Wsp, a_src0.reshape(1, D_MID), a_dst0.reshape(1, D_MID),
      a_src1.reshape(1, D_MID), a_dst1.reshape(1, D_MID))


def _rden_kernel(dp_ref, r_ref):
    r_ref[...] = 1.0 / (dp_ref[0] + dp_ref[1] + 1e-16)


def _rden(dpart):
    return pl.pallas_call(
        _rden_kernel,
        grid=(N // BN,),
        in_specs=[pl.BlockSpec((2, BN, 16), lambda i: (0, i, 0))],
        out_specs=pl.BlockSpec((BN, 16), lambda i: (i, 0)),
        out_shape=jax.ShapeDtypeStruct((N, 16), F32),
    )(dpart)


def _tc_fuse_kernel(o0_ref, o1_ref, xsp_ref, b0_ref, b1_ref, bsp_ref, w_ref,
                    g_ref, bb_ref, w2_ref, a2s_ref, a2d_ref,
                    first_ref, h2c_ref, as2_ref, ad2_ref):
    x0 = jnp.concatenate([o0_ref[0], o0_ref[1]], axis=1) + b0_ref[...]
    x0 = jnp.where(x0 > 0, x0, jnp.exp(x0) - 1.0)
    x1 = jnp.concatenate([o1_ref[0], o1_ref[1]], axis=1) + b1_ref[...]
    x1 = jnp.where(x1 > 0, x1, jnp.exp(x1) - 1.0)
    w0 = w_ref[0, 0]
    w1 = w_ref[0, 1]
    xc = w0 * x0 + w1 * x1 + xsp_ref[...] + bsp_ref[...]
    mu = xc.mean(axis=1, keepdims=True)
    var = ((xc - mu) ** 2).mean(axis=1, keepdims=True)
    f = (xc - mu) / jnp.sqrt(var + 1e-5) * g_ref[...] + bb_ref[...]
    first_ref[...] = f
    h2 = jnp.dot(f, w2_ref[...], preferred_element_type=F32)
    h2c_ref[0] = h2[:, :32]
    h2c_ref[1] = h2[:, 32:]
    z = jnp.zeros((BN, 15), F32)
    as2_ref[...] = jnp.concatenate(
        [(h2 * a2s_ref[...]).sum(-1, keepdims=True), z], axis=1)
    ad2_ref[...] = jnp.concatenate(
        [(h2 * a2d_ref[...]).sum(-1, keepdims=True), z], axis=1)


def _tc_fuse(o0, o1, xsp, b0, b1, bsp, w2d, ln1_g, ln1_b, W2, a_src2, a_dst2):
    vec_mid = pl.BlockSpec((1, D_MID), lambda i: (0, 0))
    return pl.pallas_call(
        _tc_fuse_kernel,
        grid=(N // BN,),
        in_specs=[
            pl.BlockSpec((2, BN, 128), lambda i: (0, i, 0)),
            pl.BlockSpec((2, BN, 128), lambda i: (0, i, 0)),
            pl.BlockSpec((BN, D_MID), lambda i: (i, 0)),
            vec_mid, vec_mid, vec_mid,
            pl.BlockSpec(memory_space=pltpu.SMEM),
            vec_mid, vec_mid,
            pl.BlockSpec((D_MID, D_OUT), lambda i: (0, 0)),
            pl.BlockSpec((1, D_OUT), lambda i: (0, 0)),
            pl.BlockSpec((1, D_OUT), lambda i: (0, 0)),
        ],
        out_specs=[
            pl.BlockSpec((BN, D_MID), lambda i: (i, 0)),
            pl.BlockSpec((2, BN, 32), lambda i: (0, i, 0)),
            pl.BlockSpec((BN, 16), lambda i: (i, 0)),
            pl.BlockSpec((BN, 16), lambda i: (i, 0)),
        ],
        out_shape=[
            jax.ShapeDtypeStruct((N, D_MID), F32),
            jax.ShapeDtypeStruct((2, N, 32), F32),
            jax.ShapeDtypeStruct((N, 16), F32),
            jax.ShapeDtypeStruct((N, 16), F32),
        ],
    )(o0, o1, xsp, b0.reshape(1, D_MID), b1.reshape(1, D_MID),
      bsp.reshape(1, D_MID), w2d, ln1_g.reshape(1, D_MID),
      ln1_b.reshape(1, D_MID), W2, a_src2.reshape(1, D_OUT),
      a_dst2.reshape(1, D_OUT))


def _tc_final_kernel(o2_ref, first_ref, wfp_ref, b2_ref, bfp_ref, g2_ref,
                     bb2_ref, out_ref):
    o = jnp.concatenate([o2_ref[0], o2_ref[1]], axis=1) + b2_ref[...]
    o = o + jnp.dot(first_ref[...], wfp_ref[...],
                    preferred_element_type=F32) + bfp_ref[...]
    mu = o.mean(axis=1, keepdims=True)
    var = ((o - mu) ** 2).mean(axis=1, keepdims=True)
    out_ref[...] = (o - mu) / jnp.sqrt(var + 1e-5) * g2_ref[...] + bb2_ref[...]


def _tc_final(o2, first, Wfp, b2, bfp, ln2_g, ln2_b):
    vec = pl.BlockSpec((1, D_OUT), lambda i: (0, 0))
    return pl.pallas_call(
        _tc_final_kernel,
        grid=(N // BN,),
        in_specs=[
            pl.BlockSpec((2, BN, 32), lambda i: (0, i, 0)),
            pl.BlockSpec((BN, D_MID), lambda i: (i, 0)),
            pl.BlockSpec((D_MID, D_OUT), lambda i: (0, 0)),
            vec, vec, vec, vec,
        ],
        out_specs=pl.BlockSpec((BN, D_OUT), lambda i: (i, 0)),
        out_shape=jax.ShapeDtypeStruct((N, D_OUT), F32),
    )(o2, first, Wfp, b2.reshape(1, D_OUT), bfp.reshape(1, D_OUT),
      ln2_g.reshape(1, D_OUT), ln2_b.reshape(1, D_OUT))


def _edge_phase(edge_index, asrc, adst, hflat, ncols):
    src = edge_index[0]
    dst = edge_index[1]
    a = asrc[src, :] + adst[dst, :]
    alpha = jnp.where(a >= 0, a, 0.2 * a)
    ea = jnp.exp(alpha)
    denom = jax.ops.segment_sum(ea, dst, num_segments=N)
    dpart = jnp.stack([denom, jnp.zeros_like(denom)], axis=0)
    rden = _rden(dpart)
    coef = ea * rden[dst]
    nheads = ncols // 32
    outs = []
    for c in range(2):
        h = hflat[c * N:(c + 1) * N][src]
        if ncols == 128:
            cc = coef[:, c * nheads:(c + 1) * nheads]
        else:
            cc = coef[:, :1]
        m = h.reshape(-1, nheads, 32) * cc[:, :, None]
        outs.append(
            jax.ops.segment_sum(m.reshape(-1, ncols), dst, num_segments=N))
    return jnp.stack(outs, axis=0)


def kernel(x, edge_index, edge_index2, W0, a_src0, a_dst0, b0, W1, a_src1,
           a_dst1, b1, W2, a_src2, a_dst2, b2, hop_attention, ln1_g, ln1_b,
           ln2_g, ln2_b, Wsp, bsp, Wfp, bfp):
    w2d = _hop_weights(hop_attention)
    h0c, h1c, xsp, as0, ad0, as1, ad1 = _tc_pre(
        x, W0, W1, Wsp, a_src0, a_dst0, a_src1, a_dst1)
    h0f = h0c.reshape(2 * N, 128)
    h1f = h1c.reshape(2 * N, 128)

    o0 = _edge_phase(edge_index, as0, ad0, h0f, 128)
    o1 = _edge_phase(edge_index2, as1, ad1, h1f, 128)

    first, h2c, as2, ad2 = _tc_fuse(o0, o1, xsp, b0, b1, bsp, w2d, ln1_g,
                                    ln1_b, W2, a_src2, a_dst2)
    h2f = h2c.reshape(2 * N, 32)
    o2 = _edge_phase(edge_index, as2, ad2, h2f, 64)

    out = _tc_final(o2, first, Wfp, b2, bfp, ln2_g, ln2_b)
    return (out, w2d.reshape(2))


# speedup vs baseline: 5.6223x; 5.6223x over previous
import functools

import jax
import jax.numpy as jnp
from jax import lax
from jax.experimental import pallas as pl
from jax.experimental.pallas import tpu as pltpu
from jax.experimental.pallas import tpu_sc as plsc

N = 10000
E = 320000
E2 = 640000
D_IN = 128
HEADS = 8
C_HID = 32
D_MID = 256
D_OUT = 64
BN = 1000
F32 = jnp.float32


def _w_kernel(a_ref, w_ref):
    a = a_ref[...]
    m = jnp.max(a)
    e = jnp.exp(a - m)
    w_ref[...] = e / jnp.sum(e)


def _hop_weights(hop_attention):
    w2d = pl.pallas_call(
        _w_kernel,
        out_shape=jax.ShapeDtypeStruct((1, 2), F32),
    )(hop_attention.reshape(1, 2))
    return w2d


def _tc_pre_kernel(x_ref, w0_ref, w1_ref, wsp_ref, a0s_ref, a0d_ref, a1s_ref,
                   a1d_ref, h0c_ref, h1c_ref, xsp_ref, as0_ref, ad0_ref,
                   as1_ref, ad1_ref):
    xb = x_ref[...]
    z = jnp.zeros((BN, 8), F32)

    def hop(w_ref, asr, adr, hc_ref, aso_ref, ado_ref):
        h = jnp.dot(xb, w_ref[...], preferred_element_type=F32)
        hc_ref[0] = h[:, :128]
        hc_ref[1] = h[:, 128:]
        asv = (h * asr[...]).reshape(BN, HEADS, C_HID).sum(-1)
        adv = (h * adr[...]).reshape(BN, HEADS, C_HID).sum(-1)
        aso_ref[...] = jnp.concatenate([asv, z], axis=1)
        ado_ref[...] = jnp.concatenate([adv, z], axis=1)

    hop(w0_ref, a0s_ref, a0d_ref, h0c_ref, as0_ref, ad0_ref)
    hop(w1_ref, a1s_ref, a1d_ref, h1c_ref, as1_ref, ad1_ref)
    xsp_ref[...] = jnp.dot(xb, wsp_ref[...], preferred_element_type=F32)


def _tc_pre(x, W0, W1, Wsp, a_src0, a_dst0, a_src1, a_dst1):
    full_w = pl.BlockSpec((D_IN, D_MID), lambda i: (0, 0))
    full_a = pl.BlockSpec((1, D_MID), lambda i: (0, 0))
    row = pl.BlockSpec((BN, D_IN), lambda i: (i, 0))
    chunk_o = pl.BlockSpec((2, BN, 128), lambda i: (0, i, 0))
    tbl_o = pl.BlockSpec((BN, 16), lambda i: (i, 0))
    return pl.pallas_call(
        _tc_pre_kernel,
        grid=(N // BN,),
        in_specs=[row, full_w, full_w, full_w, full_a, full_a, full_a, full_a],
        out_specs=[chunk_o, chunk_o, pl.BlockSpec((BN, D_MID), lambda i: (i, 0)),
                   tbl_o, tbl_o, tbl_o, tbl_o],
        out_shape=[
            jax.ShapeDtypeStruct((2, N, 128), F32),
            jax.ShapeDtypeStruct((2, N, 128), F32),
            jax.ShapeDtypeStruct((N, D_MID), F32),
            jax.ShapeDtypeStruct((N, 16), F32),
            jax.ShapeDtypeStruct((N, 16), F32),
            jax.ShapeDtypeStruct((N, 16), F32),
            jax.ShapeDtypeStruct((N, 16), F32),
        ],
    )(x, W0, W1, Wsp, a_src0.reshape(1, D_MID), a_dst0.reshape(1, D_MID),
      a_src1.reshape(1, D_MID), a_dst1.reshape(1, D_MID))


def _rden_kernel(dp_ref, r_ref):
    r_ref[...] = 1.0 / (dp_ref[0] + dp_ref[1] + 1e-16)


def _rden(dpart):
    return pl.pallas_call(
        _rden_kernel,
        grid=(N // BN,),
        in_specs=[pl.BlockSpec((2, BN, 16), lambda i: (0, i, 0))],
        out_specs=pl.BlockSpec((BN, 16), lambda i: (i, 0)),
        out_shape=jax.ShapeDtypeStruct((N, 16), F32),
    )(dpart)


def _tc_fuse_kernel(o0_ref, o1_ref, xsp_ref, b0_ref, b1_ref, bsp_ref, w_ref,
                    g_ref, bb_ref, w2_ref, a2s_ref, a2d_ref,
                    first_ref, h2c_ref, as2_ref, ad2_ref):
    x0 = jnp.concatenate([o0_ref[0], o0_ref[1]], axis=1) + b0_ref[...]
    x0 = jnp.where(x0 > 0, x0, jnp.exp(x0) - 1.0)
    x1 = jnp.concatenate([o1_ref[0], o1_ref[1]], axis=1) + b1_ref[...]
    x1 = jnp.where(x1 > 0, x1, jnp.exp(x1) - 1.0)
    w0 = w_ref[0, 0]
    w1 = w_ref[0, 1]
    xc = w0 * x0 + w1 * x1 + xsp_ref[...] + bsp_ref[...]
    mu = xc.mean(axis=1, keepdims=True)
    var = ((xc - mu) ** 2).mean(axis=1, keepdims=True)
    f = (xc - mu) / jnp.sqrt(var + 1e-5) * g_ref[...] + bb_ref[...]
    first_ref[...] = f
    h2 = jnp.dot(f, w2_ref[...], preferred_element_type=F32)
    h2c_ref[0] = h2[:, :32]
    h2c_ref[1] = h2[:, 32:]
    z = jnp.zeros((BN, 15), F32)
    as2_ref[...] = jnp.concatenate(
        [(h2 * a2s_ref[...]).sum(-1, keepdims=True), z], axis=1)
    ad2_ref[...] = jnp.concatenate(
        [(h2 * a2d_ref[...]).sum(-1, keepdims=True), z], axis=1)


def _tc_fuse(o0, o1, xsp, b0, b1, bsp, w2d, ln1_g, ln1_b, W2, a_src2, a_dst2):
    vec_mid = pl.BlockSpec((1, D_MID), lambda i: (0, 0))
    return pl.pallas_call(
        _tc_fuse_kernel,
        grid=(N // BN,),
        in_specs=[
            pl.BlockSpec((2, BN, 128), lambda i: (0, i, 0)),
            pl.BlockSpec((2, BN, 128), lambda i: (0, i, 0)),
            pl.BlockSpec((BN, D_MID), lambda i: (i, 0)),
            vec_mid, vec_mid, vec_mid,
            pl.BlockSpec(memory_space=pltpu.SMEM),
            vec_mid, vec_mid,
            pl.BlockSpec((D_MID, D_OUT), lambda i: (0, 0)),
            pl.BlockSpec((1, D_OUT), lambda i: (0, 0)),
            pl.BlockSpec((1, D_OUT), lambda i: (0, 0)),
        ],
        out_specs=[
            pl.BlockSpec((BN, D_MID), lambda i: (i, 0)),
            pl.BlockSpec((2, BN, 32), lambda i: (0, i, 0)),
            pl.BlockSpec((BN, 16), lambda i: (i, 0)),
            pl.BlockSpec((BN, 16), lambda i: (i, 0)),
        ],
        out_shape=[
            jax.ShapeDtypeStruct((N, D_MID), F32),
            jax.ShapeDtypeStruct((2, N, 32), F32),
            jax.ShapeDtypeStruct((N, 16), F32),
            jax.ShapeDtypeStruct((N, 16), F32),
        ],
    )(o0, o1, xsp, b0.reshape(1, D_MID), b1.reshape(1, D_MID),
      bsp.reshape(1, D_MID), w2d, ln1_g.reshape(1, D_MID),
      ln1_b.reshape(1, D_MID), W2, a_src2.reshape(1, D_OUT),
      a_dst2.reshape(1, D_OUT))


def _tc_final_kernel(o2_ref, first_ref, wfp_ref, b2_ref, bfp_ref, g2_ref,
                     bb2_ref, out_ref):
    o = jnp.concatenate([o2_ref[0], o2_ref[1]], axis=1) + b2_ref[...]
    o = o + jnp.dot(first_ref[...], wfp_ref[...],
                    preferred_element_type=F32) + bfp_ref[...]
    mu = o.mean(axis=1, keepdims=True)
    var = ((o - mu) ** 2).mean(axis=1, keepdims=True)
    out_ref[...] = (o - mu) / jnp.sqrt(var + 1e-5) * g2_ref[...] + bb2_ref[...]


def _tc_final(o2, first, Wfp, b2, bfp, ln2_g, ln2_b):
    vec = pl.BlockSpec((1, D_OUT), lambda i: (0, 0))
    return pl.pallas_call(
        _tc_final_kernel,
        grid=(N // BN,),
        in_specs=[
            pl.BlockSpec((2, BN, 32), lambda i: (0, i, 0)),
            pl.BlockSpec((BN, D_MID), lambda i: (i, 0)),
            pl.BlockSpec((D_MID, D_OUT), lambda i: (0, 0)),
            vec, vec, vec, vec,
        ],
        out_specs=pl.BlockSpec((BN, D_OUT), lambda i: (i, 0)),
        out_shape=jax.ShapeDtypeStruct((N, D_OUT), F32),
    )(o2, first, Wfp, b2.reshape(1, D_OUT), bfp.reshape(1, D_OUT),
      ln2_g.reshape(1, D_OUT), ln2_b.reshape(1, D_OUT))


def _edge_phase(edge_index, asrc, adst, hflat, ncols):
    src = edge_index[0]
    dst = edge_index[1]
    a = asrc[src, :] + adst[dst, :]
    alpha = jnp.where(a >= 0, a, 0.2 * a)
    ea = jnp.exp(alpha)
    denom = jax.ops.segment_sum(ea, dst, num_segments=N)
    dpart = jnp.stack([denom, jnp.zeros_like(denom)], axis=0)
    rden = _rden(dpart)
    coef = ea * rden[dst]
    nheads = ncols // 32
    outs = []
    for c in range(2):
        h = hflat[c * N:(c + 1) * N][src]
        if ncols == 128:
            cc = coef[:, c * nheads:(c + 1) * nheads]
        else:
            cc = coef[:, :1]
        m = h.reshape(-1, nheads, 32) * cc[:, :, None]
        outs.append(
            jax.ops.segment_sum(m.reshape(-1, ncols), dst, num_segments=N))
    return jnp.stack(outs, axis=0)


def kernel(x, edge_index, edge_index2, W0, a_src0, a_dst0, b0, W1, a_src1,
           a_dst1, b1, W2, a_src2, a_dst2, b2, hop_attention, ln1_g, ln1_b,
           ln2_g, ln2_b, Wsp, bsp, Wfp, bfp):
    w2d = _hop_weights(hop_attention)
    h0c, h1c, xsp, as0, ad0, as1, ad1 = _tc_pre(
        x, W0, W1, Wsp, a_src0, a_dst0, a_src1, a_dst1)
    h0f = h0c.reshape(2 * N, 128)
    h1f = h1c.reshape(2 * N, 128)

    o0 = _edge_phase(edge_index, as0, ad0, h0f, 128)
    o1 = _edge_phase(edge_index2, as1, ad1, h1f, 128)

    first, h2c, as2, ad2 = _tc_fuse(o0, o1, xsp, b0, b1, bsp, w2d, ln1_g,
                                    ln1_b, W2, a_src2, a_dst2)
    h2f = h2c.reshape(2 * N, 32)
    o2 = _edge_phase(edge_index, as2, ad2, h2f, 32)

    out = _tc_final(o2, first, Wfp, b2, bfp, ln2_g, ln2_b)
    return (out, w2d.reshape(2))


# speedup vs baseline: 26.2701x; 4.6725x over previous
import functools

import jax
import jax.numpy as jnp
from jax import lax
from jax.experimental import pallas as pl
from jax.experimental.pallas import tpu as pltpu
from jax.experimental.pallas import tpu_sc as plsc

N = 10000
E = 320000
E2 = 640000
D_IN = 128
HEADS = 8
C_HID = 32
D_MID = 256
D_OUT = 64
BN = 1000
F32 = jnp.float32


def _w_kernel(a_ref, w_ref):
    a = a_ref[...]
    m = jnp.max(a)
    e = jnp.exp(a - m)
    w_ref[...] = e / jnp.sum(e)


def _hop_weights(hop_attention):
    w2d = pl.pallas_call(
        _w_kernel,
        out_shape=jax.ShapeDtypeStruct((1, 2), F32),
    )(hop_attention.reshape(1, 2))
    return w2d


def _tc_pre_kernel(x_ref, w0_ref, w1_ref, wsp_ref, a0s_ref, a0d_ref, a1s_ref,
                   a1d_ref, h0c_ref, h1c_ref, xsp_ref, as0_ref, ad0_ref,
                   as1_ref, ad1_ref):
    xb = x_ref[...]
    z = jnp.zeros((BN, 8), F32)

    def hop(w_ref, asr, adr, hc_ref, aso_ref, ado_ref):
        h = jnp.dot(xb, w_ref[...], preferred_element_type=F32)
        hc_ref[0] = h[:, :128]
        hc_ref[1] = h[:, 128:]
        asv = (h * asr[...]).reshape(BN, HEADS, C_HID).sum(-1)
        adv = (h * adr[...]).reshape(BN, HEADS, C_HID).sum(-1)
        aso_ref[...] = jnp.concatenate([asv, z], axis=1)
        ado_ref[...] = jnp.concatenate([adv, z], axis=1)

    hop(w0_ref, a0s_ref, a0d_ref, h0c_ref, as0_ref, ad0_ref)
    hop(w1_ref, a1s_ref, a1d_ref, h1c_ref, as1_ref, ad1_ref)
    xsp_ref[...] = jnp.dot(xb, wsp_ref[...], preferred_element_type=F32)


def _tc_pre(x, W0, W1, Wsp, a_src0, a_dst0, a_src1, a_dst1):
    full_w = pl.BlockSpec((D_IN, D_MID), lambda i: (0, 0))
    full_a = pl.BlockSpec((1, D_MID), lambda i: (0, 0))
    row = pl.BlockSpec((BN, D_IN), lambda i: (i, 0))
    chunk_o = pl.BlockSpec((2, BN, 128), lambda i: (0, i, 0))
    tbl_o = pl.BlockSpec((BN, 16), lambda i: (i, 0))
    return pl.pallas_call(
        _tc_pre_kernel,
        grid=(N // BN,),
        in_specs=[row, full_w, full_w, full_w, full_a, full_a, full_a, full_a],
        out_specs=[chunk_o, chunk_o, pl.BlockSpec((BN, D_MID), lambda i: (i, 0)),
                   tbl_o, tbl_o, tbl_o, tbl_o],
        out_shape=[
            jax.ShapeDtypeStruct((2, N, 128), F32),
            jax.ShapeDtypeStruct((2, N, 128), F32),
            jax.ShapeDtypeStruct((N, D_MID), F32),
            jax.ShapeDtypeStruct((N, 16), F32),
            jax.ShapeDtypeStruct((N, 16), F32),
            jax.ShapeDtypeStruct((N, 16), F32),
            jax.ShapeDtypeStruct((N, 16), F32),
        ],
    )(x, W0, W1, Wsp, a_src0.reshape(1, D_MID), a_dst0.reshape(1, D_MID),
      a_src1.reshape(1, D_MID), a_dst1.reshape(1, D_MID))


def _rden_kernel(dp_ref, r_ref):
    r_ref[...] = 1.0 / (dp_ref[0] + dp_ref[1] + 1e-16)


def _rden(dpart):
    return pl.pallas_call(
        _rden_kernel,
        grid=(N // BN,),
        in_specs=[pl.BlockSpec((2, BN, 16), lambda i: (0, i, 0))],
        out_specs=pl.BlockSpec((BN, 16), lambda i: (i, 0)),
        out_shape=jax.ShapeDtypeStruct((N, 16), F32),
    )(dpart)


def _tc_fuse_kernel(o0_ref, o1_ref, xsp_ref, b0_ref, b1_ref, bsp_ref, w_ref,
                    g_ref, bb_ref, w2_ref, a2s_ref, a2d_ref,
                    first_ref, h2c_ref, as2_ref, ad2_ref):
    x0 = jnp.concatenate([o0_ref[0], o0_ref[1]], axis=1) + b0_ref[...]
    x0 = jnp.where(x0 > 0, x0, jnp.exp(x0) - 1.0)
    x1 = jnp.concatenate([o1_ref[0], o1_ref[1]], axis=1) + b1_ref[...]
    x1 = jnp.where(x1 > 0, x1, jnp.exp(x1) - 1.0)
    w0 = w_ref[0, 0]
    w1 = w_ref[0, 1]
    xc = w0 * x0 + w1 * x1 + xsp_ref[...] + bsp_ref[...]
    mu = xc.mean(axis=1, keepdims=True)
    var = ((xc - mu) ** 2).mean(axis=1, keepdims=True)
    f = (xc - mu) / jnp.sqrt(var + 1e-5) * g_ref[...] + bb_ref[...]
    first_ref[...] = f
    h2 = jnp.dot(f, w2_ref[...], preferred_element_type=F32)
    h2c_ref[0] = h2[:, :32]
    h2c_ref[1] = h2[:, 32:]
    z = jnp.zeros((BN, 15), F32)
    as2_ref[...] = jnp.concatenate(
        [(h2 * a2s_ref[...]).sum(-1, keepdims=True), z], axis=1)
    ad2_ref[...] = jnp.concatenate(
        [(h2 * a2d_ref[...]).sum(-1, keepdims=True), z], axis=1)


def _tc_fuse(o0, o1, xsp, b0, b1, bsp, w2d, ln1_g, ln1_b, W2, a_src2, a_dst2):
    vec_mid = pl.BlockSpec((1, D_MID), lambda i: (0, 0))
    return pl.pallas_call(
        _tc_fuse_kernel,
        grid=(N // BN,),
        in_specs=[
            pl.BlockSpec((2, BN, 128), lambda i: (0, i, 0)),
            pl.BlockSpec((2, BN, 128), lambda i: (0, i, 0)),
            pl.BlockSpec((BN, D_MID), lambda i: (i, 0)),
            vec_mid, vec_mid, vec_mid,
            pl.BlockSpec(memory_space=pltpu.SMEM),
            vec_mid, vec_mid,
            pl.BlockSpec((D_MID, D_OUT), lambda i: (0, 0)),
            pl.BlockSpec((1, D_OUT), lambda i: (0, 0)),
            pl.BlockSpec((1, D_OUT), lambda i: (0, 0)),
        ],
        out_specs=[
            pl.BlockSpec((BN, D_MID), lambda i: (i, 0)),
            pl.BlockSpec((2, BN, 32), lambda i: (0, i, 0)),
            pl.BlockSpec((BN, 16), lambda i: (i, 0)),
            pl.BlockSpec((BN, 16), lambda i: (i, 0)),
        ],
        out_shape=[
            jax.ShapeDtypeStruct((N, D_MID), F32),
            jax.ShapeDtypeStruct((2, N, 32), F32),
            jax.ShapeDtypeStruct((N, 16), F32),
            jax.ShapeDtypeStruct((N, 16), F32),
        ],
    )(o0, o1, xsp, b0.reshape(1, D_MID), b1.reshape(1, D_MID),
      bsp.reshape(1, D_MID), w2d, ln1_g.reshape(1, D_MID),
      ln1_b.reshape(1, D_MID), W2, a_src2.reshape(1, D_OUT),
      a_dst2.reshape(1, D_OUT))


def _tc_final_kernel(o2_ref, first_ref, wfp_ref, b2_ref, bfp_ref, g2_ref,
                     bb2_ref, out_ref):
    o = jnp.concatenate([o2_ref[0], o2_ref[1]], axis=1) + b2_ref[...]
    o = o + jnp.dot(first_ref[...], wfp_ref[...],
                    preferred_element_type=F32) + bfp_ref[...]
    mu = o.mean(axis=1, keepdims=True)
    var = ((o - mu) ** 2).mean(axis=1, keepdims=True)
    out_ref[...] = (o - mu) / jnp.sqrt(var + 1e-5) * g2_ref[...] + bb2_ref[...]


def _tc_final(o2, first, Wfp, b2, bfp, ln2_g, ln2_b):
    vec = pl.BlockSpec((1, D_OUT), lambda i: (0, 0))
    return pl.pallas_call(
        _tc_final_kernel,
        grid=(N // BN,),
        in_specs=[
            pl.BlockSpec((2, BN, 32), lambda i: (0, i, 0)),
            pl.BlockSpec((BN, D_MID), lambda i: (i, 0)),
            pl.BlockSpec((D_MID, D_OUT), lambda i: (0, 0)),
            vec, vec, vec, vec,
        ],
        out_specs=pl.BlockSpec((BN, D_OUT), lambda i: (i, 0)),
        out_shape=jax.ShapeDtypeStruct((N, D_OUT), F32),
    )(o2, first, Wfp, b2.reshape(1, D_OUT), bfp.reshape(1, D_OUT),
      ln2_g.reshape(1, D_OUT), ln2_b.reshape(1, D_OUT))


@functools.cache
def _mesh():
    return plsc.VectorSubcoreMesh(core_axis_name="c", subcore_axis_name="s",
                                  num_cores=2, num_subcores=16)
_NROW = N // 16
_EB = 80


def _row_ranges(s):
    return (s * 624, 624)


def _sc_pass1(edge_index, asrc, adst):
    el = edge_index.shape[1]
    per_w = el // 32
    zer = jnp.zeros((N, 16), F32)

    def body(src_ref, dst_ref, asrc_ref, adst_ref, zer_ref, ea_ref, dpart_ref,
             dacc, sidx_v, didx_v, gs_v, gd_v, ea_v, sem):
        c = lax.axis_index("c")
        s = lax.axis_index("s")
        wid = c * 16 + s
        r0, rn = _row_ranges(s)
        pltpu.sync_copy(zer_ref.at[pl.ds(r0, rn)], dacc.at[pl.ds(r0, rn)])
        @pl.when(s == 0)
        def _():
            pltpu.sync_copy(zer_ref.at[pl.ds(9984, 16)],
                            dacc.at[pl.ds(9984, 16)])
        plsc.subcore_barrier()
        base = wid * per_w

        def blk(j, _):
            off = base + j * _EB
            pltpu.sync_copy(src_ref.at[pl.ds(off, _EB)], sidx_v)
            pltpu.sync_copy(dst_ref.at[pl.ds(off, _EB)], didx_v.at[0])
            pltpu.async_copy(asrc_ref.at[sidx_v], gs_v, sem).wait()
            pltpu.async_copy(adst_ref.at[didx_v.at[0]], gd_v, sem).wait()
            for i in range(_EB):
                a = gs_v[i, :] + gd_v[i, :]
                al = jnp.where(a >= 0, a, 0.2 * a)
                ea_v[i, :] = jnp.exp(al)
            pltpu.sync_copy(ea_v, ea_ref.at[pl.ds(off, _EB)])
            pltpu.sync_copy(ea_v, dacc.at[didx_v.at[0]], add=True)
            return ()

        lax.fori_loop(0, per_w // _EB, blk, ())
        plsc.subcore_barrier()
        pltpu.sync_copy(dacc.at[pl.ds(r0, rn)],
                        dpart_ref.at[pl.ds(c * N + r0, rn)])
        @pl.when(s == 0)
        def _():
            pltpu.sync_copy(dacc.at[pl.ds(9984, 16)],
                            dpart_ref.at[pl.ds(c * N + 9984, 16)])

    fn = pl.kernel(
        body,
        out_type=[
            jax.ShapeDtypeStruct((el, 16), F32),
            jax.ShapeDtypeStruct((2 * N, 16), F32),
        ],
        mesh=_mesh(),
        scratch_types=[
            pltpu.VMEM_SHARED((N, 16), F32),
            pltpu.VMEM((_EB,), jnp.int32),
            pltpu.VMEM((1, _EB), jnp.int32),
            pltpu.VMEM((_EB, 16), F32),
            pltpu.VMEM((_EB, 16), F32),
            pltpu.VMEM((_EB, 16), F32),
            pltpu.SemaphoreType.DMA,
        ],
        compiler_params=pltpu.CompilerParams(use_tc_tiling_on_sc=False),
    )
    return fn(edge_index[0], edge_index[1], asrc, adst, zer)


def _sc_pass2(edge_index, hflat, ea, rden, ncols, hpc):
    el = edge_index.shape[1]
    per_t = el // 16
    nvr = ncols // 16
    zer = jnp.zeros((N, ncols), F32)

    def body(src_ref, dst_ref, h_ref, ea_ref, rden_ref, zer_ref, o_ref,
             acc, sidx_v, didx_v, hbuf, ea_v, rd_v, msg, sem):
        c = lax.axis_index("c")
        s = lax.axis_index("s")
        cn = c * N
        loff = c * hpc
        r0, rn = _row_ranges(s)
        pltpu.sync_copy(zer_ref.at[pl.ds(r0, rn)], acc.at[pl.ds(r0, rn)])
        @pl.when(s == 0)
        def _():
            pltpu.sync_copy(zer_ref.at[pl.ds(9984, 16)],
                            acc.at[pl.ds(9984, 16)])
        plsc.subcore_barrier()
        base = s * per_t

        def blk(j, _):
            off = base + j * _EB
            pltpu.sync_copy(src_ref.at[pl.ds(off, _EB)], sidx_v)
            pltpu.sync_copy(dst_ref.at[pl.ds(off, _EB)], didx_v.at[0])
            for t in range(_EB // 16):
                sidx_v[pl.ds(t * 16, 16)] = sidx_v[pl.ds(t * 16, 16)] + cn
            pltpu.async_copy(h_ref.at[sidx_v], hbuf, sem).wait()
            pltpu.sync_copy(ea_ref.at[pl.ds(off, _EB)], ea_v)
            pltpu.async_copy(rden_ref.at[didx_v.at[0]], rd_v, sem).wait()
            zi = jnp.zeros((16,), jnp.int32)
            for i in range(_EB):
                cv = ea_v[i, :] * rd_v[i, :]
                for k in range(nvr):
                    coefv = cv.at[zi + (loff + k // 2)].get(
                        mode="promise_in_bounds")
                    msg[i, pl.ds(k * 16, 16)] = (
                        hbuf[i, pl.ds(k * 16, 16)] * coefv)
            pltpu.sync_copy(msg, acc.at[didx_v.at[0]], add=True)
            return ()

        lax.fori_loop(0, per_t // _EB, blk, ())
        plsc.subcore_barrier()
        pltpu.sync_copy(acc.at[pl.ds(r0, rn)], o_ref.at[pl.ds(cn + r0, rn)])
        @pl.when(s == 0)
        def _():
            pltpu.sync_copy(acc.at[pl.ds(9984, 16)],
                            o_ref.at[pl.ds(cn + 9984, 16)])

    fn = pl.kernel(
        body,
        out_type=jax.ShapeDtypeStruct((2 * N, ncols), F32),
        mesh=_mesh(),
        scratch_types=[
            pltpu.VMEM_SHARED((N, ncols), F32),
            pltpu.VMEM((_EB,), jnp.int32),
            pltpu.VMEM((1, _EB), jnp.int32),
            pltpu.VMEM((_EB, ncols), F32),
            pltpu.VMEM((_EB, 16), F32),
            pltpu.VMEM((_EB, 16), F32),
            pltpu.VMEM((_EB, ncols), F32),
            pltpu.SemaphoreType.DMA,
        ],
        compiler_params=pltpu.CompilerParams(use_tc_tiling_on_sc=False),
    )
    return fn(edge_index[0], edge_index[1], hflat, ea, rden, zer)


def _edge_phase(edge_index, asrc, adst, hflat, ncols):
    ea, dpart = _sc_pass1(edge_index, asrc, adst)
    rden = _rden(dpart.reshape(2, N, 16))
    hpc = 4 if ncols == 128 else 0
    o = _sc_pass2(edge_index, hflat, ea, rden, ncols, hpc)
    return o.reshape(2, N, ncols)


def kernel(x, edge_index, edge_index2, W0, a_src0, a_dst0, b0, W1, a_src1,
           a_dst1, b1, W2, a_src2, a_dst2, b2, hop_attention, ln1_g, ln1_b,
           ln2_g, ln2_b, Wsp, bsp, Wfp, bfp):
    w2d = _hop_weights(hop_attention)
    h0c, h1c, xsp, as0, ad0, as1, ad1 = _tc_pre(
        x, W0, W1, Wsp, a_src0, a_dst0, a_src1, a_dst1)
    h0f = h0c.reshape(2 * N, 128)
    h1f = h1c.reshape(2 * N, 128)

    o0 = _edge_phase(edge_index, as0, ad0, h0f, 128)
    o1 = _edge_phase(edge_index2, as1, ad1, h1f, 128)

    first, h2c, as2, ad2 = _tc_fuse(o0, o1, xsp, b0, b1, bsp, w2d, ln1_g,
                                    ln1_b, W2, a_src2, a_dst2)
    h2f = h2c.reshape(2 * N, 32)
    o2 = _edge_phase(edge_index, as2, ad2, h2f, 32)

    out = _tc_final(o2, first, Wfp, b2, bfp, ln2_g, ln2_b)
    return (out, w2d.reshape(2))


# speedup vs baseline: 49.3199x; 1.8774x over previous
import functools

import jax
import jax.numpy as jnp
from jax import lax
from jax.experimental import pallas as pl
from jax.experimental.pallas import tpu as pltpu
from jax.experimental.pallas import tpu_sc as plsc

N = 10000
E = 320000
E2 = 640000
D_IN = 128
HEADS = 8
C_HID = 32
D_MID = 256
D_OUT = 64
BN = 1000
F32 = jnp.float32


def _w_kernel(a_ref, w_ref):
    a = a_ref[...]
    m = jnp.max(a)
    e = jnp.exp(a - m)
    w_ref[...] = e / jnp.sum(e)


def _hop_weights(hop_attention):
    w2d = pl.pallas_call(
        _w_kernel,
        out_shape=jax.ShapeDtypeStruct((1, 2), F32),
    )(hop_attention.reshape(1, 2))
    return w2d


def _tc_pre_kernel(x_ref, w0_ref, w1_ref, wsp_ref, a0s_ref, a0d_ref, a1s_ref,
                   a1d_ref, h0c_ref, h1c_ref, xsp_ref, as0_ref, ad0_ref,
                   as1_ref, ad1_ref):
    xb = x_ref[...]
    z = jnp.zeros((BN, 8), F32)

    def hop(w_ref, asr, adr, hc_ref, aso_ref, ado_ref):
        h = jnp.dot(xb, w_ref[...], preferred_element_type=F32)
        hc_ref[0] = h[:, :128]
        hc_ref[1] = h[:, 128:]
        asv = (h * asr[...]).reshape(BN, HEADS, C_HID).sum(-1)
        adv = (h * adr[...]).reshape(BN, HEADS, C_HID).sum(-1)
        aso_ref[...] = jnp.concatenate([asv, z], axis=1)
        ado_ref[...] = jnp.concatenate([adv, z], axis=1)

    hop(w0_ref, a0s_ref, a0d_ref, h0c_ref, as0_ref, ad0_ref)
    hop(w1_ref, a1s_ref, a1d_ref, h1c_ref, as1_ref, ad1_ref)
    xsp_ref[...] = jnp.dot(xb, wsp_ref[...], preferred_element_type=F32)


def _tc_pre(x, W0, W1, Wsp, a_src0, a_dst0, a_src1, a_dst1):
    full_w = pl.BlockSpec((D_IN, D_MID), lambda i: (0, 0))
    full_a = pl.BlockSpec((1, D_MID), lambda i: (0, 0))
    row = pl.BlockSpec((BN, D_IN), lambda i: (i, 0))
    chunk_o = pl.BlockSpec((2, BN, 128), lambda i: (0, i, 0))
    tbl_o = pl.BlockSpec((BN, 16), lambda i: (i, 0))
    return pl.pallas_call(
        _tc_pre_kernel,
        grid=(N // BN,),
        in_specs=[row, full_w, full_w, full_w, full_a, full_a, full_a, full_a],
        out_specs=[chunk_o, chunk_o, pl.BlockSpec((BN, D_MID), lambda i: (i, 0)),
                   tbl_o, tbl_o, tbl_o, tbl_o],
        out_shape=[
            jax.ShapeDtypeStruct((2, N, 128), F32),
            jax.ShapeDtypeStruct((2, N, 128), F32),
            jax.ShapeDtypeStruct((N, D_MID), F32),
            jax.ShapeDtypeStruct((N, 16), F32),
            jax.ShapeDtypeStruct((N, 16), F32),
            jax.ShapeDtypeStruct((N, 16), F32),
            jax.ShapeDtypeStruct((N, 16), F32),
        ],
    )(x, W0, W1, Wsp, a_src0.reshape(1, D_MID), a_dst0.reshape(1, D_MID),
      a_src1.reshape(1, D_MID), a_dst1.reshape(1, D_MID))


def _rden_kernel(dp_ref, r_ref):
    r_ref[...] = 1.0 / (dp_ref[0] + dp_ref[1] + 1e-16)


def _rden(dpart):
    return pl.pallas_call(
        _rden_kernel,
        grid=(N // BN,),
        in_specs=[pl.BlockSpec((2, BN, 16), lambda i: (0, i, 0))],
        out_specs=pl.BlockSpec((BN, 16), lambda i: (i, 0)),
        out_shape=jax.ShapeDtypeStruct((N, 16), F32),
    )(dpart)


def _tc_fuse_kernel(o0_ref, o1_ref, xsp_ref, b0_ref, b1_ref, bsp_ref, w_ref,
                    g_ref, bb_ref, w2_ref, a2s_ref, a2d_ref,
                    first_ref, h2c_ref, as2_ref, ad2_ref):
    x0 = jnp.concatenate([o0_ref[0], o0_ref[1]], axis=1) + b0_ref[...]
    x0 = jnp.where(x0 > 0, x0, jnp.exp(x0) - 1.0)
    x1 = jnp.concatenate([o1_ref[0], o1_ref[1]], axis=1) + b1_ref[...]
    x1 = jnp.where(x1 > 0, x1, jnp.exp(x1) - 1.0)
    w0 = w_ref[0, 0]
    w1 = w_ref[0, 1]
    xc = w0 * x0 + w1 * x1 + xsp_ref[...] + bsp_ref[...]
    mu = xc.mean(axis=1, keepdims=True)
    var = ((xc - mu) ** 2).mean(axis=1, keepdims=True)
    f = (xc - mu) / jnp.sqrt(var + 1e-5) * g_ref[...] + bb_ref[...]
    first_ref[...] = f
    h2 = jnp.dot(f, w2_ref[...], preferred_element_type=F32)
    h2c_ref[0] = h2[:, :32]
    h2c_ref[1] = h2[:, 32:]
    z = jnp.zeros((BN, 15), F32)
    as2_ref[...] = jnp.concatenate(
        [(h2 * a2s_ref[...]).sum(-1, keepdims=True), z], axis=1)
    ad2_ref[...] = jnp.concatenate(
        [(h2 * a2d_ref[...]).sum(-1, keepdims=True), z], axis=1)


def _tc_fuse(o0, o1, xsp, b0, b1, bsp, w2d, ln1_g, ln1_b, W2, a_src2, a_dst2):
    vec_mid = pl.BlockSpec((1, D_MID), lambda i: (0, 0))
    return pl.pallas_call(
        _tc_fuse_kernel,
        grid=(N // BN,),
        in_specs=[
            pl.BlockSpec((2, BN, 128), lambda i: (0, i, 0)),
            pl.BlockSpec((2, BN, 128), lambda i: (0, i, 0)),
            pl.BlockSpec((BN, D_MID), lambda i: (i, 0)),
            vec_mid, vec_mid, vec_mid,
            pl.BlockSpec(memory_space=pltpu.SMEM),
            vec_mid, vec_mid,
            pl.BlockSpec((D_MID, D_OUT), lambda i: (0, 0)),
            pl.BlockSpec((1, D_OUT), lambda i: (0, 0)),
            pl.BlockSpec((1, D_OUT), lambda i: (0, 0)),
        ],
        out_specs=[
            pl.BlockSpec((BN, D_MID), lambda i: (i, 0)),
            pl.BlockSpec((2, BN, 32), lambda i: (0, i, 0)),
            pl.BlockSpec((BN, 16), lambda i: (i, 0)),
            pl.BlockSpec((BN, 16), lambda i: (i, 0)),
        ],
        out_shape=[
            jax.ShapeDtypeStruct((N, D_MID), F32),
            jax.ShapeDtypeStruct((2, N, 32), F32),
            jax.ShapeDtypeStruct((N, 16), F32),
            jax.ShapeDtypeStruct((N, 16), F32),
        ],
    )(o0, o1, xsp, b0.reshape(1, D_MID), b1.reshape(1, D_MID),
      bsp.reshape(1, D_MID), w2d, ln1_g.reshape(1, D_MID),
      ln1_b.reshape(1, D_MID), W2, a_src2.reshape(1, D_OUT),
      a_dst2.reshape(1, D_OUT))


def _tc_final_kernel(o2_ref, first_ref, wfp_ref, b2_ref, bfp_ref, g2_ref,
                     bb2_ref, out_ref):
    o = jnp.concatenate([o2_ref[0], o2_ref[1]], axis=1) + b2_ref[...]
    o = o + jnp.dot(first_ref[...], wfp_ref[...],
                    preferred_element_type=F32) + bfp_ref[...]
    mu = o.mean(axis=1, keepdims=True)
    var = ((o - mu) ** 2).mean(axis=1, keepdims=True)
    out_ref[...] = (o - mu) / jnp.sqrt(var + 1e-5) * g2_ref[...] + bb2_ref[...]


def _tc_final(o2, first, Wfp, b2, bfp, ln2_g, ln2_b):
    vec = pl.BlockSpec((1, D_OUT), lambda i: (0, 0))
    return pl.pallas_call(
        _tc_final_kernel,
        grid=(N // BN,),
        in_specs=[
            pl.BlockSpec((2, BN, 32), lambda i: (0, i, 0)),
            pl.BlockSpec((BN, D_MID), lambda i: (i, 0)),
            pl.BlockSpec((D_MID, D_OUT), lambda i: (0, 0)),
            vec, vec, vec, vec,
        ],
        out_specs=pl.BlockSpec((BN, D_OUT), lambda i: (i, 0)),
        out_shape=jax.ShapeDtypeStruct((N, D_OUT), F32),
    )(o2, first, Wfp, b2.reshape(1, D_OUT), bfp.reshape(1, D_OUT),
      ln2_g.reshape(1, D_OUT), ln2_b.reshape(1, D_OUT))


@functools.cache
def _mesh():
    return plsc.VectorSubcoreMesh(core_axis_name="c", subcore_axis_name="s",
                                  num_cores=2, num_subcores=16)
_NROW = N // 16
_EB = 80
_CH = 8000


def _row_ranges(s):
    return (s * 624, 624)


def _sc_pass1(edge_index, asrc, adst):
    el = edge_index.shape[1]
    per_w = el // 32
    zer = jnp.zeros((N, 16), F32)
    chunks = []
    off = 0
    while off < per_w:
        ln = min(_CH, per_w - off)
        chunks.append((off, ln))
        off += ln

    def body(src_ref, dst_ref, asrc_ref, adst_ref, zer_ref, ea_ref, dpart_ref,
             dacc, sxall, dxall, sidx_v, didx_v, gs_v, gd_v, ea_v, sem):
        c = lax.axis_index("c")
        s = lax.axis_index("s")
        wid = c * 16 + s
        r0, rn = _row_ranges(s)
        pltpu.sync_copy(zer_ref.at[pl.ds(r0, rn)], dacc.at[pl.ds(r0, rn)])
        @pl.when(s == 0)
        def _():
            pltpu.sync_copy(zer_ref.at[pl.ds(9984, 16)],
                            dacc.at[pl.ds(9984, 16)])
        plsc.subcore_barrier()
        base = wid * per_w

        for chs, chn in chunks:
            pltpu.sync_copy(src_ref.at[pl.ds(base + chs, chn)],
                            sxall.at[pl.ds(0, chn)])
            pltpu.sync_copy(dst_ref.at[pl.ds(base + chs, chn)],
                            dxall.at[pl.ds(0, chn)])

            def blk(j, _):
                for t in range(_EB // 16):
                    sl = pl.ds(j * _EB + t * 16, 16)
                    sidx_v[pl.ds(t * 16, 16)] = sxall[sl]
                    didx_v[0, pl.ds(t * 16, 16)] = dxall[sl]
                d1 = pltpu.async_copy(asrc_ref.at[sidx_v], gs_v, sem)
                d2 = pltpu.async_copy(adst_ref.at[didx_v.at[0]], gd_v, sem)
                d1.wait()
                d2.wait()
                for i in range(_EB):
                    a = gs_v[i, :] + gd_v[i, :]
                    al = jnp.where(a >= 0, a, 0.2 * a)
                    ea_v[i, :] = jnp.exp(al)
                pltpu.sync_copy(
                    ea_v, ea_ref.at[pl.ds(base + chs + j * _EB, _EB)])
                pltpu.sync_copy(ea_v, dacc.at[didx_v.at[0]], add=True)
                return ()

            lax.fori_loop(0, chn // _EB, blk, ())
        plsc.subcore_barrier()
        pltpu.sync_copy(dacc.at[pl.ds(r0, rn)],
                        dpart_ref.at[pl.ds(c * N + r0, rn)])
        @pl.when(s == 0)
        def _():
            pltpu.sync_copy(dacc.at[pl.ds(9984, 16)],
                            dpart_ref.at[pl.ds(c * N + 9984, 16)])

    fn = pl.kernel(
        body,
        out_type=[
            jax.ShapeDtypeStruct((el, 16), F32),
            jax.ShapeDtypeStruct((2 * N, 16), F32),
        ],
        mesh=_mesh(),
        scratch_types=[
            pltpu.VMEM_SHARED((N, 16), F32),
            pltpu.VMEM((_CH,), jnp.int32),
            pltpu.VMEM((_CH,), jnp.int32),
            pltpu.VMEM((_EB,), jnp.int32),
            pltpu.VMEM((1, _EB), jnp.int32),
            pltpu.VMEM((_EB, 16), F32),
            pltpu.VMEM((_EB, 16), F32),
            pltpu.VMEM((_EB, 16), F32),
            pltpu.SemaphoreType.DMA,
        ],
        compiler_params=pltpu.CompilerParams(use_tc_tiling_on_sc=False),
    )
    return fn(edge_index[0], edge_index[1], asrc, adst, zer)


def _sc_pass2(edge_index, hflat, ea, rden, ncols, hpc):
    el = edge_index.shape[1]
    per_t = el // 16
    nvr = ncols // 16
    zer = jnp.zeros((N, ncols), F32)
    chunks = []
    off = 0
    while off < per_t:
        ln = min(_CH, per_t - off)
        chunks.append((off, ln))
        off += ln

    def body(src_ref, dst_ref, h_ref, ea_ref, rden_ref, zer_ref, o_ref,
             acc, sxall, dxall, sidx_v, didx_v, hbuf, ea_v, rd_v, msg, sem):
        c = lax.axis_index("c")
        s = lax.axis_index("s")
        cn = c * N
        loff = c * hpc
        r0, rn = _row_ranges(s)
        pltpu.sync_copy(zer_ref.at[pl.ds(r0, rn)], acc.at[pl.ds(r0, rn)])
        @pl.when(s == 0)
        def _():
            pltpu.sync_copy(zer_ref.at[pl.ds(9984, 16)],
                            acc.at[pl.ds(9984, 16)])
        plsc.subcore_barrier()
        base = s * per_t

        for chs, chn in chunks:
            pltpu.sync_copy(src_ref.at[pl.ds(base + chs, chn)],
                            sxall.at[pl.ds(0, chn)])
            pltpu.sync_copy(dst_ref.at[pl.ds(base + chs, chn)],
                            dxall.at[pl.ds(0, chn)])

            def blk(j, _):
                for t in range(_EB // 16):
                    sl = pl.ds(j * _EB + t * 16, 16)
                    sidx_v[pl.ds(t * 16, 16)] = sxall[sl] + cn
                    didx_v[0, pl.ds(t * 16, 16)] = dxall[sl]
                d1 = pltpu.async_copy(h_ref.at[sidx_v], hbuf, sem)
                d2 = pltpu.async_copy(
                    ea_ref.at[pl.ds(base + chs + j * _EB, _EB)], ea_v, sem)
                d3 = pltpu.async_copy(rden_ref.at[didx_v.at[0]], rd_v, sem)
                d1.wait()
                d2.wait()
                d3.wait()
                zi = jnp.zeros((16,), jnp.int32)
                for i in range(_EB):
                    cv = ea_v[i, :] * rd_v[i, :]
                    for k in range(nvr):
                        coefv = cv.at[zi + (loff + k // 2)].get(
                            mode="promise_in_bounds")
                        msg[i, pl.ds(k * 16, 16)] = (
                            hbuf[i, pl.ds(k * 16, 16)] * coefv)
                pltpu.sync_copy(msg, acc.at[didx_v.at[0]], add=True)
                return ()

            lax.fori_loop(0, chn // _EB, blk, ())
        plsc.subcore_barrier()
        pltpu.sync_copy(acc.at[pl.ds(r0, rn)], o_ref.at[pl.ds(cn + r0, rn)])
        @pl.when(s == 0)
        def _():
            pltpu.sync_copy(acc.at[pl.ds(9984, 16)],
                            o_ref.at[pl.ds(cn + 9984, 16)])

    fn = pl.kernel(
        body,
        out_type=jax.ShapeDtypeStruct((2 * N, ncols), F32),
        mesh=_mesh(),
        scratch_types=[
            pltpu.VMEM_SHARED((N, ncols), F32),
            pltpu.VMEM((_CH,), jnp.int32),
            pltpu.VMEM((_CH,), jnp.int32),
            pltpu.VMEM((_EB,), jnp.int32),
            pltpu.VMEM((1, _EB), jnp.int32),
            pltpu.VMEM((_EB, ncols), F32),
            pltpu.VMEM((_EB, 16), F32),
            pltpu.VMEM((_EB, 16), F32),
            pltpu.VMEM((_EB, ncols), F32),
            pltpu.SemaphoreType.DMA,
        ],
        compiler_params=pltpu.CompilerParams(use_tc_tiling_on_sc=False),
    )
    return fn(edge_index[0], edge_index[1], hflat, ea, rden, zer)


def _edge_phase(edge_index, asrc, adst, hflat, ncols):
    ea, dpart = _sc_pass1(edge_index, asrc, adst)
    rden = _rden(dpart.reshape(2, N, 16))
    hpc = 4 if ncols == 128 else 0
    o = _sc_pass2(edge_index, hflat, ea, rden, ncols, hpc)
    return o.reshape(2, N, ncols)


def kernel(x, edge_index, edge_index2, W0, a_src0, a_dst0, b0, W1, a_src1,
           a_dst1, b1, W2, a_src2, a_dst2, b2, hop_attention, ln1_g, ln1_b,
           ln2_g, ln2_b, Wsp, bsp, Wfp, bfp):
    w2d = _hop_weights(hop_attention)
    h0c, h1c, xsp, as0, ad0, as1, ad1 = _tc_pre(
        x, W0, W1, Wsp, a_src0, a_dst0, a_src1, a_dst1)
    h0f = h0c.reshape(2 * N, 128)
    h1f = h1c.reshape(2 * N, 128)

    o0 = _edge_phase(edge_index, as0, ad0, h0f, 128)
    o1 = _edge_phase(edge_index2, as1, ad1, h1f, 128)

    first, h2c, as2, ad2 = _tc_fuse(o0, o1, xsp, b0, b1, bsp, w2d, ln1_g,
                                    ln1_b, W2, a_src2, a_dst2)
    h2f = h2c.reshape(2 * N, 32)
    o2 = _edge_phase(edge_index, as2, ad2, h2f, 32)

    out = _tc_final(o2, first, Wfp, b2, bfp, ln2_g, ln2_b)
    return (out, w2d.reshape(2))
